# baseline (speedup 1.0000x reference)
import functools

import jax
import jax.numpy as jnp
from jax import lax
from jax.experimental import pallas as pl
from jax.experimental.pallas import tpu as pltpu
from jax.experimental.pallas import tpu_sc as plsc

N_NODES = 100000
N_EDGES = 3200000
DE = 16
DX = 128

NW = 32
NRANGE = NW * 2
R = 1568
R2 = 2 * R
S = 1576
NPAD = NRANGE * R
C = 1280
NCHUNK = N_EDGES // C
G = 256
U = 5
NB = 4
FLUSH = 1024
CAP = FLUSH + 2 * C + G
ACAP = CAP + G + 16
BLK = 1024
STCAP = BLK + CAP + BLK + 32
SPCAP = N_EDGES + BLK


def _lane_splat(v, j):
    return lax.gather(
        v, jnp.full((16, 1), j, jnp.int32),
        lax.GatherDimensionNumbers(offset_dims=(), collapsed_slice_dims=(0,),
                                   start_index_map=(0,)),
        (1,), mode=lax.GatherScatterMode.PROMISE_IN_BOUNDS)


def _sc_body(dst_hbm, fe_hbm, sm_hbm, ct_hbm, mi_hbm, ma_hbm, spe_hbm,
             spd_hbm,
             dstbuf, pkbuf, eidbuf, dlbuf, stbe, stbd, rows,
             asum, acnt, amin, amax, smem,
             semc0, semc1, semg0, semg1, semg2, semg3):
    cid = lax.axis_index("c")
    sid = lax.axis_index("s")
    wid = sid * 2 + cid
    lo01 = wid * R2
    iota = lax.iota(jnp.int32, 16)
    zero = jnp.zeros((16,), jnp.float32)
    pinf = jnp.full((16,), jnp.inf, jnp.float32)
    ninf = jnp.full((16,), -jnp.inf, jnp.float32)
    onev = jnp.ones((16,), jnp.float32)
    padR = jnp.full((16,), R, jnp.int32)
    pad0 = jnp.zeros((16,), jnp.int32)
    semc = (semc0, semc1)
    semg = (semg0, semg1, semg2, semg3)

    def chunk_start(ci, b):
        pltpu.async_copy(dst_hbm.at[pl.ds(ci * C, C)],
                         dstbuf.at[pl.ds(b * C, C)], semc[b])

    def chunk_wait(b):
        pltpu.make_async_copy(dst_hbm.at[pl.ds(0, C)],
                              dstbuf.at[pl.ds(b * C, C)], semc[b]).wait()

    def gather_start(g, b):
        pltpu.async_copy(fe_hbm.at[eidbuf.at[pl.ds(g * G, G)]],
                         rows.at[pl.ds(b * G, G)], semg[b])

    def gather_wait(b):
        pltpu.make_async_copy(fe_hbm.at[eidbuf.at[pl.ds(0, G)]],
                              rows.at[pl.ds(b * G, G)], semg[b]).wait()

    def consume(ngroups):
        for b in range(NB):
            @pl.when(b < ngroups)
            def _():
                gather_start(b, b)

        def quad(q, _):
            for b in range(NB):
                g = q * NB + b

                @pl.when(g < ngroups)
                def _():
                    gather_wait(b)

                    def sgroup(s, _):
                        dlv = dlbuf[pl.ds(g * G + s * 16, 16)]
                        rsp0 = jnp.full((16,), 1, jnp.int32) \
                            * (b * G + s * 16)
                        for j in range(16):
                            dsp = _lane_splat(dlv, j)
                            row = plsc.load_gather(rows, [rsp0 + j, iota])
                            plsc.addupdate_scatter(asum, [dsp, iota], row)
                            m0 = plsc.load_gather(amin, [dsp, iota])
                            plsc.store_scatter(amin, [dsp, iota],
                                               jnp.minimum(m0, row))
                            x0 = plsc.load_gather(amax, [dsp, iota])
                            plsc.store_scatter(amax, [dsp, iota],
                                               jnp.maximum(x0, row))
                        return 0

                    lax.fori_loop(0, G // 16, sgroup, 0)

                    @pl.when(g + NB < ngroups)
                    def _():
                        gather_start(g + NB, b)
            return 0

        lax.fori_loop(0, (ngroups + NB - 1) // NB, quad, 0)

    def spill_blocks(nfull, nblk):
        def spill(k, _):
            pltpu.sync_copy(stbe.at[pl.ds(k * BLK, BLK)],
                            spe_hbm.at[wid, pl.ds((nblk + k) * BLK, BLK)])
            pltpu.sync_copy(stbd.at[pl.ds(k * BLK, BLK)],
                            spd_hbm.at[wid, pl.ds((nblk + k) * BLK, BLK)])
            return 0

        lax.fori_loop(0, nfull, spill, 0)

    def flush(ptr, fb):
        fbv = jnp.full((16,), 1, jnp.int32) * fb
        ptrv = jnp.full((16,), 1, jnp.int32) * ptr
        pB0 = smem[0]

        def unpack(i, c):
            pA, pB = c
            pk = pkbuf[pl.ds(i * 16, 16)]
            valid = (i * 16 + iota) < ptrv
            dl01 = pk & 0xFFF
            eid = lax.shift_right_logical(pk, 12) + fbv
            inA = dl01 < R
            mA = valid & inA
            mB = valid & (~inA)
            nA = plsc.all_reduce_population_count(mA)[0]
            nB = plsc.all_reduce_population_count(mB)[0]
            plsc.store_compressed(dlbuf.at[pl.ds(pA, 16)], dl01, mask=mA)
            plsc.store_compressed(eidbuf.at[pl.ds(pA, 16)], eid, mask=mA)
            plsc.addupdate_scatter(acnt, [dl01], onev, mask=mA)
            plsc.store_compressed(stbd.at[pl.ds(pB, 16)], dl01 - R,
                                  mask=mB)
            plsc.store_compressed(stbe.at[pl.ds(pB, 16)], eid, mask=mB)
            return (pA + nA, pB + nB)

        ptrA, ptrB = lax.fori_loop(0, (ptr + 15) // 16, unpack,
                                   (jnp.int32(0), pB0))

        for k in range(G // 16):
            dlbuf[pl.ds(ptrA + k * 16, 16)] = padR
            eidbuf[pl.ds(ptrA + k * 16, 16)] = pad0
        consume((ptrA + (G - 1)) // G)

        nfull = ptrB // BLK
        nblk = smem[1]
        spill_blocks(nfull, nblk)
        tail = ptrB - nfull * BLK

        @pl.when(nfull > 0)
        def _():
            def rebase(i, _):
                stbe[pl.ds(i * 16, 16)] = stbe[pl.ds(nfull * BLK + i * 16,
                                                     16)]
                stbd[pl.ds(i * 16, 16)] = stbd[pl.ds(nfull * BLK + i * 16,
                                                     16)]
                return 0

            lax.fori_loop(0, (tail + 15) // 16, rebase, 0)

        smem[0] = tail
        smem[1] = nblk + nfull

    def initacc(i, _):
        asum[i] = zero
        amin[i] = pinf
        amax[i] = ninf
        return 0

    def initcnt(i, _):
        acnt[pl.ds(i * 16, 16)] = zero
        return 0

    lax.fori_loop(0, S, initacc, 0)
    lax.fori_loop(0, S // 16, initcnt, 0)
    smem[0] = 0
    smem[1] = 0

    chunk_start(0, 0)
    iotash = lax.shift_left(iota, 12)

    def chunkpair(cp, carry):
        ptr, fb = carry
        for b in range(2):
            ci = cp * 2 + b
            chunk_wait(b)

            @pl.when(ci + 1 < NCHUNK)
            def _():
                chunk_start(ci + 1, 1 - b)

            base = b * C

            def scanbody(t, ptr):
                tb = base + t * (16 * U)
                pks, masks, ns = [], [], []
                for u in range(U):
                    dvec = dstbuf[pl.ds(tb + u * 16, 16)]
                    dl = dvec - lo01
                    mask = dl.astype(jnp.uint32) < jnp.uint32(R2)
                    ebase = (ci * C - fb + t * (16 * U) + u * 16) << 12
                    pk = dl + (jnp.full((16,), 1, jnp.int32) * ebase) \
                        + iotash
                    pks.append(pk)
                    masks.append(mask)
                    ns.append(plsc.all_reduce_population_count(mask))
                for u in range(U):
                    plsc.store_compressed(pkbuf.at[pl.ds(ptr, 16)],
                                          pks[u], mask=masks[u])
                    ptr = ptr + ns[u][0]
                return ptr

            ptr = lax.fori_loop(0, C // 16 // U, scanbody, ptr)

        do_flush = ((ptr >= FLUSH) | (cp == NCHUNK // 2 - 1)
                    | ((cp & 127) == 127))

        @pl.when(do_flush)
        def _():
            flush(ptr, fb)

        nfb = (cp + 1) * (2 * C)
        return (jnp.where(do_flush, 0, ptr),
                jnp.where(do_flush, nfb, fb))

    lax.fori_loop(0, NCHUNK // 2, chunkpair,
                  (jnp.int32(0), jnp.int32(0)))

    pltpu.sync_copy(asum.at[pl.ds(0, R)], sm_hbm.at[pl.ds(lo01, R)])
    pltpu.sync_copy(acnt.at[pl.ds(0, R)], ct_hbm.at[pl.ds(lo01, R)])
    pltpu.sync_copy(amin.at[pl.ds(0, R)], mi_hbm.at[pl.ds(lo01, R)])
    pltpu.sync_copy(amax.at[pl.ds(0, R)], ma_hbm.at[pl.ds(lo01, R)])

    tail = smem[0]

    @pl.when(tail > 0)
    def _():
        npadv = (BLK - tail + 15) // 16

        def padtail(k, _):
            stbd[pl.ds(tail + k * 16, 16)] = padR
            stbe[pl.ds(tail + k * 16, 16)] = pad0
            return 0

        lax.fori_loop(0, npadv, padtail, 0)
        spill_blocks(1, smem[1])
        smem[1] = smem[1] + 1

    lax.fori_loop(0, S, initacc, 0)
    lax.fori_loop(0, S // 16, initcnt, 0)

    nblkB = smem[1]

    def bblock(blk, _):
        pltpu.sync_copy(spe_hbm.at[wid, pl.ds(blk * BLK, BLK)],
                        eidbuf.at[pl.ds(0, BLK)])
        pltpu.sync_copy(spd_hbm.at[wid, pl.ds(blk * BLK, BLK)],
                        dlbuf.at[pl.ds(0, BLK)])

        def cntb(i, _):
            dlv = dlbuf[pl.ds(i * 16, 16)]
            plsc.addupdate_scatter(acnt, [dlv], onev)
            return 0

        lax.fori_loop(0, BLK // 16, cntb, 0)
        consume(BLK // G)
        return 0

    lax.fori_loop(0, nblkB, bblock, 0)

    lo2 = lo01 + R
    pltpu.sync_copy(asum.at[pl.ds(0, R)], sm_hbm.at[pl.ds(lo2, R)])
    pltpu.sync_copy(acnt.at[pl.ds(0, R)], ct_hbm.at[pl.ds(lo2, R)])
    pltpu.sync_copy(amin.at[pl.ds(0, R)], mi_hbm.at[pl.ds(lo2, R)])
    pltpu.sync_copy(amax.at[pl.ds(0, R)], ma_hbm.at[pl.ds(lo2, R)])


def _sc_reduce(dst, fe):
    mesh = plsc.VectorSubcoreMesh(core_axis_name="c", subcore_axis_name="s",
                                  num_cores=2, num_subcores=16)
    f = pl.kernel(
        _sc_body,
        out_type=[jax.ShapeDtypeStruct((NPAD, DE), jnp.float32),
                  jax.ShapeDtypeStruct((NPAD,), jnp.float32),
                  jax.ShapeDtypeStruct((NPAD, DE), jnp.float32),
                  jax.ShapeDtypeStruct((NPAD, DE), jnp.float32),
                  jax.ShapeDtypeStruct((NW, SPCAP), jnp.int32),
                  jax.ShapeDtypeStruct((NW, SPCAP), jnp.int32)],
        mesh=mesh,
        compiler_params=pltpu.CompilerParams(needs_layout_passes=False,
                                             use_tc_tiling_on_sc=False),
        scratch_types=[
            pltpu.VMEM((2 * C,), jnp.int32),
            pltpu.VMEM((CAP + 16,), jnp.int32),
            pltpu.VMEM((ACAP,), jnp.int32),
            pltpu.VMEM((ACAP,), jnp.int32),
            pltpu.VMEM((STCAP,), jnp.int32),
            pltpu.VMEM((STCAP,), jnp.int32),
            pltpu.VMEM((NB * G, DE), jnp.float32),
            pltpu.VMEM((S, DE), jnp.float32),
            pltpu.VMEM((S,), jnp.float32),
            pltpu.VMEM((S, DE), jnp.float32),
            pltpu.VMEM((S, DE), jnp.float32),
            pltpu.SMEM((8,), jnp.int32),
            pltpu.SemaphoreType.DMA,
            pltpu.SemaphoreType.DMA,
            pltpu.SemaphoreType.DMA,
            pltpu.SemaphoreType.DMA,
            pltpu.SemaphoreType.DMA,
            pltpu.SemaphoreType.DMA,
        ],
    )
    return f(dst, fe)


BT = 2048


def _tc_body(sm_ref, ct_ref, mi_ref, ma_ref, wm_ref, wi_ref, wa_ref, b_ref,
             o_ref):
    cv = ct_ref[...][:, None]
    has = cv > 0.0
    me = jnp.where(has, sm_ref[...] / jnp.maximum(cv, 1.0), 0.0)
    mi = jnp.where(has, mi_ref[...], 0.0)
    ma = jnp.where(has, ma_ref[...], 0.0)
    acc = jnp.dot(me, wm_ref[...], preferred_element_type=jnp.float32)
    acc += jnp.dot(mi, wi_ref[...], preferred_element_type=jnp.float32)
    acc += jnp.dot(ma, wa_ref[...], preferred_element_type=jnp.float32)
    o_ref[...] = acc + b_ref[...]


def _tc_linear(sm, ct, mi, ma, wm, wi, wa, b2):
    nblk = NPAD // BT
    zspec = pl.BlockSpec((BT, DE), lambda i: (i, 0))
    cspec = pl.BlockSpec((BT,), lambda i: (i,))
    wspec = pl.BlockSpec((DE, DX), lambda i: (0, 0))
    bspec = pl.BlockSpec((1, DX), lambda i: (0, 0))
    return pl.pallas_call(
        _tc_body,
        grid=(nblk,),
        in_specs=[zspec, cspec, zspec, zspec, wspec, wspec, wspec, bspec],
        out_specs=pl.BlockSpec((BT, DX), lambda i: (i, 0)),
        out_shape=jax.ShapeDtypeStruct((NPAD, DX), jnp.float32),
    )(sm, ct, mi, ma, wm, wi, wa, b2)


def kernel(fe, edge_index, W, b):
    dst = edge_index[1]
    sm, ct, mi, ma, _spe, _spd = _sc_reduce(dst, fe)
    wm = W[:, :DE].T
    wi = W[:, DE:2 * DE].T
    wa = W[:, 2 * DE:].T
    out = _tc_linear(sm, ct, mi, ma, wm, wi, wa, b.reshape(1, DX))
    return out[:N_NODES]


# speedup vs baseline: 1.1931x; 1.1931x over previous
import functools

import jax
import jax.numpy as jnp
from jax import lax
from jax.experimental import pallas as pl
from jax.experimental.pallas import tpu as pltpu
from jax.experimental.pallas import tpu_sc as plsc

N_NODES = 100000
N_EDGES = 3200000
DE = 16
DX = 128

NW = 32
NPASS = 2
NRANGE = NW * NPASS
R = 1568
S = 1576
NPAD = NRANGE * R
C = 6400
NCHUNK = N_EDGES // C
G = 256
U = 5
NB = 4
FLUSH = 1024
CAP = FLUSH + C + G


def _lane_splat(v, j):
    return lax.gather(
        v, jnp.full((16, 1), j, jnp.int32),
        lax.GatherDimensionNumbers(offset_dims=(), collapsed_slice_dims=(0,),
                                   start_index_map=(0,)),
        (1,), mode=lax.GatherScatterMode.PROMISE_IN_BOUNDS)


def _sc_body(dst_hbm, fe_hbm, sm_hbm, ct_hbm, mi_hbm, ma_hbm,
             dstbuf, pkbuf, eidbuf, dlbuf, rows, asum, acnt, amin, amax,
             semc0, semc1, semg0, semg1, semg2, semg3):
    cid = lax.axis_index("c")
    sid = lax.axis_index("s")
    wid = sid * 2 + cid
    iota = lax.iota(jnp.int32, 16)
    zero = jnp.zeros((16,), jnp.float32)
    pinf = jnp.full((16,), jnp.inf, jnp.float32)
    ninf = jnp.full((16,), -jnp.inf, jnp.float32)
    onev = jnp.ones((16,), jnp.float32)
    semc = (semc0, semc1)
    semg = (semg0, semg1, semg2, semg3)

    def chunk_start(ci, b):
        pltpu.async_copy(dst_hbm.at[pl.ds(ci * C, C)],
                         dstbuf.at[pl.ds(b * C, C)], semc[b])

    def chunk_wait(b):
        pltpu.make_async_copy(dst_hbm.at[pl.ds(0, C)],
                              dstbuf.at[pl.ds(b * C, C)], semc[b]).wait()

    def gather_start(g, b):
        pltpu.async_copy(fe_hbm.at[eidbuf.at[pl.ds(g * G, G)]],
                         rows.at[pl.ds(b * G, G)], semg[b])

    def gather_wait(b):
        pltpu.make_async_copy(fe_hbm.at[eidbuf.at[pl.ds(0, G)]],
                              rows.at[pl.ds(b * G, G)], semg[b]).wait()

    def flush(ptr, fb):
        padv = jnp.full((16,), R, jnp.int32)
        for k in range(G // 16):
            pkbuf[pl.ds(ptr + k * 16, 16)] = padv

        ngroups = (ptr + (G - 1)) // G
        fbv = jnp.full((16,), 1, jnp.int32) * fb

        def unpack(i, _):
            pk = pkbuf[pl.ds(i * 16, 16)]
            dl = pk & 0x7FF
            dlbuf[pl.ds(i * 16, 16)] = dl
            eidbuf[pl.ds(i * 16, 16)] = (
                lax.shift_right_logical(pk, 11) + fbv)
            plsc.addupdate_scatter(acnt, [dl], onev)
            return 0

        lax.fori_loop(0, ngroups * (G // 16), unpack, 0)

        for b in range(NB):
            @pl.when(b < ngroups)
            def _():
                gather_start(b, b)

        def quad(q, _):
            for b in range(NB):
                g = q * NB + b

                @pl.when(g < ngroups)
                def _():
                    gather_wait(b)

                    def sgroup(s, _):
                        dlv = dlbuf[pl.ds(g * G + s * 16, 16)]
                        rsp0 = jnp.full((16,), 1, jnp.int32) \
                            * (b * G + s * 16)
                        for j in range(16):
                            dsp = _lane_splat(dlv, j)
                            row = plsc.load_gather(rows, [rsp0 + j, iota])
                            plsc.addupdate_scatter(asum, [dsp, iota], row)
                            m0 = plsc.load_gather(amin, [dsp, iota])
                            plsc.store_scatter(amin, [dsp, iota],
                                               jnp.minimum(m0, row))
                            x0 = plsc.load_gather(amax, [dsp, iota])
                            plsc.store_scatter(amax, [dsp, iota],
                                               jnp.maximum(x0, row))
                        return 0

                    lax.fori_loop(0, G // 16, sgroup, 0)

                    @pl.when(g + NB < ngroups)
                    def _():
                        gather_start(g + NB, b)
            return 0

        lax.fori_loop(0, (ngroups + NB - 1) // NB, quad, 0)

    def passbody(p, _):
        rid = wid * NPASS + p
        node_lo = rid * R

        def initacc(i, _):
            asum[i] = zero
            amin[i] = pinf
            amax[i] = ninf
            return 0

        lax.fori_loop(0, S, initacc, 0)

        def initcnt(i, _):
            acnt[pl.ds(i * 16, 16)] = zero
            return 0

        lax.fori_loop(0, S // 16, initcnt, 0)

        chunk_start(0, 0)
        iotash = lax.shift_left(iota, 11)

        def chunkpair(cp, carry):
            ptr, fb = carry
            for b in range(2):
                ci = cp * 2 + b
                chunk_wait(b)

                @pl.when(ci + 1 < NCHUNK)
                def _():
                    chunk_start(ci + 1, 1 - b)

                base = b * C

                def scanbody(t, ptr):
                    tb = base + t * (16 * U)
                    pks, masks, ns = [], [], []
                    for u in range(U):
                        dvec = dstbuf[pl.ds(tb + u * 16, 16)]
                        dl = dvec - node_lo
                        mask = dl.astype(jnp.uint32) < jnp.uint32(R)
                        ebase = (ci * C - fb + t * (16 * U) + u * 16) << 11
                        pk = dl + (jnp.full((16,), 1, jnp.int32) * ebase) \
                            + iotash
                        pks.append(pk)
                        masks.append(mask)
                        ns.append(plsc.all_reduce_population_count(mask))
                    for u in range(U):
                        plsc.store_compressed(pkbuf.at[pl.ds(ptr, 16)],
                                              pks[u], mask=masks[u])
                        ptr = ptr + ns[u][0]
                    return ptr

                ptr = lax.fori_loop(0, C // 16 // U, scanbody, ptr)

                do_flush = ((ptr >= FLUSH) | (ci == NCHUNK - 1)
                            | ((ci & 127) == 127))

                @pl.when(do_flush)
                def _():
                    flush(ptr, fb)

                nfb = (ci + 1) * C
                ptr = jnp.where(do_flush, 0, ptr)
                fb = jnp.where(do_flush, nfb, fb)

            return (ptr, fb)

        lax.fori_loop(0, NCHUNK // 2, chunkpair,
                      (jnp.int32(0), jnp.int32(0)))

        pltpu.sync_copy(asum.at[pl.ds(0, R)], sm_hbm.at[pl.ds(node_lo, R)])
        pltpu.sync_copy(acnt.at[pl.ds(0, R)], ct_hbm.at[pl.ds(node_lo, R)])
        pltpu.sync_copy(amin.at[pl.ds(0, R)], mi_hbm.at[pl.ds(node_lo, R)])
        pltpu.sync_copy(amax.at[pl.ds(0, R)], ma_hbm.at[pl.ds(node_lo, R)])
        return 0

    lax.fori_loop(0, NPASS, passbody, 0)


def _sc_reduce(dst, fe):
    mesh = plsc.VectorSubcoreMesh(core_axis_name="c", subcore_axis_name="s",
                                  num_cores=2, num_subcores=16)
    f = pl.kernel(
        _sc_body,
        out_type=[jax.ShapeDtypeStruct((NPAD, DE), jnp.float32),
                  jax.ShapeDtypeStruct((NPAD,), jnp.float32),
                  jax.ShapeDtypeStruct((NPAD, DE), jnp.float32),
                  jax.ShapeDtypeStruct((NPAD, DE), jnp.float32)],
        mesh=mesh,
        compiler_params=pltpu.CompilerParams(needs_layout_passes=False,
                                             use_tc_tiling_on_sc=False),
        scratch_types=[
            pltpu.VMEM((2 * C,), jnp.int32),
            pltpu.VMEM((CAP,), jnp.int32),
            pltpu.VMEM((CAP,), jnp.int32),
            pltpu.VMEM((CAP,), jnp.int32),
            pltpu.VMEM((NB * G, DE), jnp.float32),
            pltpu.VMEM((S, DE), jnp.float32),
            pltpu.VMEM((S,), jnp.float32),
            pltpu.VMEM((S, DE), jnp.float32),
            pltpu.VMEM((S, DE), jnp.float32),
            pltpu.SemaphoreType.DMA,
            pltpu.SemaphoreType.DMA,
            pltpu.SemaphoreType.DMA,
            pltpu.SemaphoreType.DMA,
            pltpu.SemaphoreType.DMA,
            pltpu.SemaphoreType.DMA,
        ],
    )
    return f(dst, fe)


BT = 2048


def _tc_body(sm_ref, ct_ref, mi_ref, ma_ref, wm_ref, wi_ref, wa_ref, b_ref,
             o_ref):
    cv = ct_ref[...][:, None]
    has = cv > 0.0
    me = jnp.where(has, sm_ref[...] / jnp.maximum(cv, 1.0), 0.0)
    mi = jnp.where(has, mi_ref[...], 0.0)
    ma = jnp.where(has, ma_ref[...], 0.0)
    acc = jnp.dot(me, wm_ref[...], preferred_element_type=jnp.float32)
    acc += jnp.dot(mi, wi_ref[...], preferred_element_type=jnp.float32)
    acc += jnp.dot(ma, wa_ref[...], preferred_element_type=jnp.float32)
    o_ref[...] = acc + b_ref[...]


def _tc_linear(sm, ct, mi, ma, wm, wi, wa, b2):
    nblk = NPAD // BT
    zspec = pl.BlockSpec((BT, DE), lambda i: (i, 0))
    cspec = pl.BlockSpec((BT,), lambda i: (i,))
    wspec = pl.BlockSpec((DE, DX), lambda i: (0, 0))
    bspec = pl.BlockSpec((1, DX), lambda i: (0, 0))
    return pl.pallas_call(
        _tc_body,
        grid=(nblk,),
        in_specs=[zspec, cspec, zspec, zspec, wspec, wspec, wspec, bspec],
        out_specs=pl.BlockSpec((BT, DX), lambda i: (i, 0)),
        out_shape=jax.ShapeDtypeStruct((NPAD, DX), jnp.float32),
    )(sm, ct, mi, ma, wm, wi, wa, b2)


def kernel(fe, edge_index, W, b):
    dst = edge_index[1]
    sm, ct, mi, ma = _sc_reduce(dst, fe)
    wm = W[:, :DE].T
    wi = W[:, DE:2 * DE].T
    wa = W[:, 2 * DE:].T
    out = _tc_linear(sm, ct, mi, ma, wm, wi, wa, b.reshape(1, DX))
    return out[:N_NODES]
